# baseline (speedup 1.0000x reference)
import functools

import jax
import jax.numpy as jnp
from jax import lax
from jax.experimental import pallas as pl
from jax.experimental.pallas import tpu as pltpu
from jax.experimental.pallas import tpu_sc as plsc

N_CORES = 2
N_SUB = 16
N_TILES = N_CORES * N_SUB
CHUNK = 128
LAT = 128
ZR = 120


def _spmm_body(n_pad, chunks_per_tile, rows_hbm, cols_hbm, vals_hbm, x_hbm,
               out_hbm, colv, rowv, valv, msg, zbuf, acc, sem):
    c = lax.axis_index("c")
    s = lax.axis_index("s")
    e_per_tile = chunks_per_tile * CHUNK
    tile_base = c * (N_SUB * e_per_tile) + s * e_per_tile

    z16 = jnp.zeros((16,), jnp.float32)

    def zb_body(k, _):
        r = k // 8
        j = (k % 8) * 16
        zbuf[r, pl.ds(j, 16)] = z16
        return 0

    lax.fori_loop(0, ZR * 8, zb_body, 0)

    rpt = n_pad // N_SUB
    row_lo = s * rpt
    off = 0
    while off < rpt:
        sz = min(ZR, rpt - off)
        pltpu.sync_copy(zbuf.at[pl.ds(0, sz)], acc.at[pl.ds(row_lo + off, sz)])
        off += sz
    plsc.subcore_barrier()

    iota16 = lax.iota(jnp.int32, 16)

    def chunk_body(k, _):
        base = tile_base + k * CHUNK
        pltpu.sync_copy(cols_hbm.at[pl.ds(base, CHUNK)], colv)
        pltpu.sync_copy(rows_hbm.at[pl.ds(base, CHUNK)], rowv)
        pltpu.sync_copy(vals_hbm.at[pl.ds(base, CHUNK)], valv)
        pltpu.async_copy(x_hbm.at[colv], msg, sem).wait()

        def g_body(g, _):
            vals16 = valv[pl.ds(g * 16, 16)]
            rowidx = iota16 + g * 16

            def c_body(cc, colidx):
                m = plsc.load_gather(msg, [rowidx, colidx])
                plsc.store_scatter(msg, [rowidx, colidx], m * vals16)
                return colidx + 1

            lax.fori_loop(0, LAT, c_body, jnp.zeros((16,), jnp.int32))
            return 0

        lax.fori_loop(0, CHUNK // 16, g_body, 0)
        pltpu.sync_copy(msg, acc.at[rowv], add=True)
        return 0

    lax.fori_loop(0, chunks_per_tile, chunk_body, 0)
    plsc.subcore_barrier()

    off = 0
    while off < rpt:
        sz = min(ZR, rpt - off)
        pltpu.sync_copy(acc.at[pl.ds(row_lo + off, sz)], zbuf.at[pl.ds(0, sz)])
        pltpu.sync_copy(zbuf.at[pl.ds(0, sz)], out_hbm.at[c, pl.ds(row_lo + off, sz)])
        off += sz


@functools.partial(jax.jit, static_argnames=("n_pad", "chunks_per_tile"))
def _spmm_partials(rows, cols, vals, x, *, n_pad, chunks_per_tile):
    mesh = plsc.VectorSubcoreMesh(core_axis_name="c", subcore_axis_name="s")
    body = functools.partial(_spmm_body, n_pad, chunks_per_tile)
    kern = pl.kernel(
        body,
        out_type=jax.ShapeDtypeStruct((N_CORES, n_pad, LAT), jnp.float32),
        mesh=mesh,
        compiler_params=pltpu.CompilerParams(needs_layout_passes=False),
        scratch_types=[
            pltpu.VMEM((CHUNK,), jnp.int32),
            pltpu.VMEM((CHUNK,), jnp.int32),
            pltpu.VMEM((CHUNK,), jnp.float32),
            pltpu.VMEM((CHUNK, LAT), jnp.float32),
            pltpu.VMEM((ZR, LAT), jnp.float32),
            pltpu.VMEM_SHARED((n_pad, LAT), jnp.float32),
            pltpu.SemaphoreType.DMA,
        ],
    )
    return kern(rows, cols, vals, x)


def _pad_edges(index, values):
    e = values.shape[0]
    ep = -(-e // (N_TILES * CHUNK)) * (N_TILES * CHUNK)
    pad = ep - e
    rows = jnp.pad(index[0], (0, pad))
    cols = jnp.pad(index[1], (0, pad))
    vals = jnp.pad(values, (0, pad))
    return rows, cols, vals, ep


def _spmm(rows, cols, vals, ep, x, n_pad):
    parts = _spmm_partials(rows, cols, vals, x, n_pad=n_pad,
                           chunks_per_tile=ep // (N_TILES * CHUNK))
    return parts[0] + parts[1]


def kernel(adj_index, adj_values, uadj_index, uadj_values, uEmbeds0, iEmbeds0):
    n_user = uEmbeds0.shape[0]
    n_item = iEmbeds0.shape[0]
    n_total = n_user + n_item
    n_pad1 = -(-n_total // (N_SUB * 8)) * (N_SUB * 8)
    n_pad2 = -(-n_user // (N_SUB * 8)) * (N_SUB * 8)

    rows1, cols1, vals1, ep1 = _pad_edges(adj_index, adj_values)
    rows2, cols2, vals2, ep2 = _pad_edges(uadj_index, uadj_values)

    e0 = jnp.concatenate([uEmbeds0, iEmbeds0], axis=0)
    e0 = jnp.pad(e0, ((0, n_pad1 - n_total), (0, 0)))
    e1 = _spmm(rows1, cols1, vals1, ep1, e0, n_pad1)
    e2 = _spmm(rows1, cols1, vals1, ep1, e1, n_pad1)
    pooled = e0 + e1 + e2

    u0 = jnp.pad(uEmbeds0, ((0, n_pad2 - n_user), (0, 0)))
    u1 = _spmm(rows2, cols2, vals2, ep2, u0, n_pad2)
    u2 = _spmm(rows2, cols2, vals2, ep2, u1, n_pad2)
    uu = u0 + u1 + u2

    ui_uEmbed = pooled[:n_user]
    ui_iEmbed = pooled[n_user:n_total]
    return (ui_uEmbed, ui_iEmbed, uu[:n_user])


# speedup vs baseline: 1.2028x; 1.2028x over previous
import functools

import jax
import jax.numpy as jnp
from jax import lax
from jax.experimental import pallas as pl
from jax.experimental.pallas import tpu as pltpu
from jax.experimental.pallas import tpu_sc as plsc

N_CORES = 2
N_SUB = 16
N_TILES = N_CORES * N_SUB
CHUNK = 128
LAT = 128
ZR = 64


def _spmm_body(n_pad, cpt, ev_hbm, x_hbm, out_hbm,
               ev0, ev1, m0, m1, zbuf, acc, g0, g1, s0, s1):
    c_ax = lax.axis_index("c")
    s_ax = lax.axis_index("s")
    tile = c_ax * N_SUB + s_ax
    chunk_lo = tile * cpt
    evs = [ev0, ev1]
    msgs = [m0, m1]
    semG = [g0, g1]
    semS = [s0, s1]

    z16 = jnp.zeros((16,), jnp.float32)

    def zb_body(k, _):
        zbuf[k // 8, pl.ds((k % 8) * 16, 16)] = z16
        return 0

    lax.fori_loop(0, ZR * 8, zb_body, 0)

    rpt = n_pad // N_SUB
    row_lo = s_ax * rpt
    off = 0
    while off < rpt:
        sz = min(ZR, rpt - off)
        pltpu.sync_copy(zbuf.at[pl.ds(0, sz)], acc.at[pl.ds(row_lo + off, sz)])
        off += sz
    plsc.subcore_barrier()

    iota16 = lax.iota(jnp.int32, 16)
    rowregs = [iota16 + 16 * g for g in range(8)]

    def scale_chunk(buf, ev):
        valregs = [plsc.bitcast(ev[2, pl.ds(16 * g, 16)], jnp.float32)
                   for g in range(8)]

        def col_body(col, _):
            colidx = jnp.full((16,), col, jnp.int32)
            for g in range(8):
                m = plsc.load_gather(buf, [rowregs[g], colidx])
                plsc.store_scatter(buf, [rowregs[g], colidx], m * valregs[g])
            return 0

        lax.fori_loop(0, LAT, col_body, 0, unroll=2)

    def process(c, b, first, last):
        b1 = 1 - b
        if not first:
            pltpu.make_async_copy(
                msgs[b1], acc.at[evs[b1].at[1]], semS[b1]).wait()
        if not last:
            pltpu.sync_copy(ev_hbm.at[chunk_lo + c + 1], evs[b1])
            pltpu.async_copy(x_hbm.at[evs[b1].at[0]], msgs[b1], semG[b1])
        pltpu.make_async_copy(x_hbm.at[evs[b].at[0]], msgs[b], semG[b]).wait()
        scale_chunk(msgs[b], evs[b])
        pltpu.async_copy(msgs[b], acc.at[evs[b].at[1]], semS[b], add=True)

    pltpu.sync_copy(ev_hbm.at[chunk_lo], ev0)
    pltpu.async_copy(x_hbm.at[ev0.at[0]], m0, g0)

    process(0, 0, first=True, last=False)

    def pair_body(p, _):
        c = 1 + 2 * p
        process(c, 1, first=False, last=False)
        process(c + 1, 0, first=False, last=False)
        return 0

    lax.fori_loop(0, (cpt - 2) // 2, pair_body, 0)
    process(cpt - 1, 1, first=False, last=True)
    pltpu.make_async_copy(msgs[1], acc.at[evs[1].at[1]], semS[1]).wait()
    plsc.subcore_barrier()

    off = 0
    while off < rpt:
        sz = min(ZR, rpt - off)
        pltpu.sync_copy(acc.at[pl.ds(row_lo + off, sz)], zbuf.at[pl.ds(0, sz)])
        pltpu.sync_copy(zbuf.at[pl.ds(0, sz)],
                        out_hbm.at[c_ax, pl.ds(row_lo + off, sz)])
        off += sz


@functools.partial(jax.jit, static_argnames=("n_pad", "cpt"))
def _spmm_partials(ev, x, *, n_pad, cpt):
    mesh = plsc.VectorSubcoreMesh(core_axis_name="c", subcore_axis_name="s")
    body = functools.partial(_spmm_body, n_pad, cpt)
    kern = pl.kernel(
        body,
        out_type=jax.ShapeDtypeStruct((N_CORES, n_pad, LAT), jnp.float32),
        mesh=mesh,
        compiler_params=pltpu.CompilerParams(needs_layout_passes=False),
        scratch_types=[
            pltpu.VMEM((3, CHUNK), jnp.int32),
            pltpu.VMEM((3, CHUNK), jnp.int32),
            pltpu.VMEM((CHUNK, LAT), jnp.float32),
            pltpu.VMEM((CHUNK, LAT), jnp.float32),
            pltpu.VMEM((ZR, LAT), jnp.float32),
            pltpu.VMEM_SHARED((n_pad, LAT), jnp.float32),
            pltpu.SemaphoreType.DMA,
            pltpu.SemaphoreType.DMA,
            pltpu.SemaphoreType.DMA,
            pltpu.SemaphoreType.DMA,
        ],
    )
    return kern(ev, x)


def _pad_edges(index, values):
    e = values.shape[0]
    blk = N_TILES * CHUNK * 2
    ep = -(-e // blk) * blk
    pad = ep - e
    cpt = ep // (N_TILES * CHUNK)
    nch = N_TILES * cpt
    cols = jnp.pad(index[1], (0, pad)).reshape(nch, 1, CHUNK)
    rows = jnp.pad(index[0], (0, pad)).reshape(nch, 1, CHUNK)
    vbits = lax.bitcast_convert_type(jnp.pad(values, (0, pad)),
                                     jnp.int32).reshape(nch, 1, CHUNK)
    ev = jnp.concatenate([cols, rows, vbits], axis=1)
    return ev, cpt


def _spmm(ev, cpt, x, n_pad):
    parts = _spmm_partials(ev, x, n_pad=n_pad, cpt=cpt)
    return parts[0] + parts[1]


def kernel(adj_index, adj_values, uadj_index, uadj_values, uEmbeds0, iEmbeds0):
    n_user = uEmbeds0.shape[0]
    n_item = iEmbeds0.shape[0]
    n_total = n_user + n_item
    n_pad1 = -(-n_total // (N_SUB * 8)) * (N_SUB * 8)
    n_pad2 = -(-n_user // (N_SUB * 8)) * (N_SUB * 8)

    ev1, cpt1 = _pad_edges(adj_index, adj_values)
    ev2, cpt2 = _pad_edges(uadj_index, uadj_values)

    e0 = jnp.concatenate([uEmbeds0, iEmbeds0], axis=0)
    e0 = jnp.pad(e0, ((0, n_pad1 - n_total), (0, 0)))
    e1 = _spmm(ev1, cpt1, e0, n_pad1)
    e2 = _spmm(ev1, cpt1, e1, n_pad1)
    pooled = e0 + e1 + e2

    u0 = jnp.pad(uEmbeds0, ((0, n_pad2 - n_user), (0, 0)))
    u1 = _spmm(ev2, cpt2, u0, n_pad2)
    u2 = _spmm(ev2, cpt2, u1, n_pad2)
    uu = u0 + u1 + u2

    ui_uEmbed = pooled[:n_user]
    ui_iEmbed = pooled[n_user:n_total]
    return (ui_uEmbed, ui_iEmbed, uu[:n_user])


# speedup vs baseline: 1.6583x; 1.3787x over previous
import functools

import jax
import jax.numpy as jnp
from jax import lax
from jax.experimental import pallas as pl
from jax.experimental.pallas import tpu as pltpu
from jax.experimental.pallas import tpu_sc as plsc

N_CORES = 2
N_SUB = 16
N_TILES = N_CORES * N_SUB
CHUNK = 128
LAT = 128
ZR = 64


def _spmm_body(n_pad, cpt, ev_hbm, x_hbm, out_hbm,
               ev0, ev1, m0, m1, zbuf, acc, g0, g1, s0, s1):
    c_ax = lax.axis_index("c")
    s_ax = lax.axis_index("s")
    tile = c_ax * N_SUB + s_ax
    chunk_lo = tile * cpt
    evs = [ev0, ev1]
    msgs = [m0, m1]
    semG = [g0, g1]
    semS = [s0, s1]

    z16 = jnp.zeros((16,), jnp.float32)

    def zb_body(k, _):
        zbuf[k // 8, pl.ds((k % 8) * 16, 16)] = z16
        return 0

    lax.fori_loop(0, ZR * 8, zb_body, 0)

    rpt = n_pad // N_SUB
    row_lo = s_ax * rpt
    off = 0
    while off < rpt:
        sz = min(ZR, rpt - off)
        pltpu.sync_copy(zbuf.at[pl.ds(0, sz)], acc.at[pl.ds(row_lo + off, sz)])
        off += sz
    plsc.subcore_barrier()

    iota16 = lax.iota(jnp.int32, 16)
    rowregs = [iota16 + 16 * g for g in range(8)]

    def scale_chunk(buf, ev):
        valregs = [plsc.bitcast(ev[2, pl.ds(16 * g, 16)], jnp.float32)
                   for g in range(8)]

        @plsc.parallel_loop(0, LAT, 1, unroll=4)
        def col_body(col):
            colidx = jnp.full((16,), col, jnp.int32)
            ms = [plsc.load_gather(buf, [rowregs[g], colidx])
                  for g in range(8)]
            for g in range(8):
                plsc.store_scatter(buf, [rowregs[g], colidx],
                                   ms[g] * valregs[g])

    def process(c, b, first, last):
        b1 = 1 - b
        if not first:
            pltpu.make_async_copy(
                msgs[b1], acc.at[evs[b1].at[1]], semS[b1]).wait()
        if not last:
            pltpu.sync_copy(ev_hbm.at[chunk_lo + c + 1], evs[b1])
            pltpu.async_copy(x_hbm.at[evs[b1].at[0]], msgs[b1], semG[b1])
        pltpu.make_async_copy(x_hbm.at[evs[b].at[0]], msgs[b], semG[b]).wait()
        scale_chunk(msgs[b], evs[b])
        pltpu.async_copy(msgs[b], acc.at[evs[b].at[1]], semS[b], add=True)

    pltpu.sync_copy(ev_hbm.at[chunk_lo], ev0)
    pltpu.async_copy(x_hbm.at[ev0.at[0]], m0, g0)

    process(0, 0, first=True, last=False)

    def pair_body(p, _):
        c = 1 + 2 * p
        process(c, 1, first=False, last=False)
        process(c + 1, 0, first=False, last=False)
        return 0

    lax.fori_loop(0, (cpt - 2) // 2, pair_body, 0)
    process(cpt - 1, 1, first=False, last=True)
    pltpu.make_async_copy(msgs[1], acc.at[evs[1].at[1]], semS[1]).wait()
    plsc.subcore_barrier()

    off = 0
    while off < rpt:
        sz = min(ZR, rpt - off)
        pltpu.sync_copy(acc.at[pl.ds(row_lo + off, sz)], zbuf.at[pl.ds(0, sz)])
        pltpu.sync_copy(zbuf.at[pl.ds(0, sz)],
                        out_hbm.at[c_ax, pl.ds(row_lo + off, sz)])
        off += sz


@functools.partial(jax.jit, static_argnames=("n_pad", "cpt"))
def _spmm_partials(ev, x, *, n_pad, cpt):
    mesh = plsc.VectorSubcoreMesh(core_axis_name="c", subcore_axis_name="s")
    body = functools.partial(_spmm_body, n_pad, cpt)
    kern = pl.kernel(
        body,
        out_type=jax.ShapeDtypeStruct((N_CORES, n_pad, LAT), jnp.float32),
        mesh=mesh,
        compiler_params=pltpu.CompilerParams(needs_layout_passes=False),
        scratch_types=[
            pltpu.VMEM((3, CHUNK), jnp.int32),
            pltpu.VMEM((3, CHUNK), jnp.int32),
            pltpu.VMEM((CHUNK, LAT), jnp.float32),
            pltpu.VMEM((CHUNK, LAT), jnp.float32),
            pltpu.VMEM((ZR, LAT), jnp.float32),
            pltpu.VMEM_SHARED((n_pad, LAT), jnp.float32),
            pltpu.SemaphoreType.DMA,
            pltpu.SemaphoreType.DMA,
            pltpu.SemaphoreType.DMA,
            pltpu.SemaphoreType.DMA,
        ],
    )
    return kern(ev, x)


def _pad_edges(index, values):
    e = values.shape[0]
    blk = N_TILES * CHUNK * 2
    ep = -(-e // blk) * blk
    pad = ep - e
    cpt = ep // (N_TILES * CHUNK)
    nch = N_TILES * cpt
    cols = jnp.pad(index[1], (0, pad)).reshape(nch, 1, CHUNK)
    rows = jnp.pad(index[0], (0, pad)).reshape(nch, 1, CHUNK)
    vbits = lax.bitcast_convert_type(jnp.pad(values, (0, pad)),
                                     jnp.int32).reshape(nch, 1, CHUNK)
    ev = jnp.concatenate([cols, rows, vbits], axis=1)
    return ev, cpt


def _spmm(ev, cpt, x, n_pad):
    parts = _spmm_partials(ev, x, n_pad=n_pad, cpt=cpt)
    return parts[0] + parts[1]


def kernel(adj_index, adj_values, uadj_index, uadj_values, uEmbeds0, iEmbeds0):
    n_user = uEmbeds0.shape[0]
    n_item = iEmbeds0.shape[0]
    n_total = n_user + n_item
    n_pad1 = -(-n_total // (N_SUB * 8)) * (N_SUB * 8)
    n_pad2 = -(-n_user // (N_SUB * 8)) * (N_SUB * 8)

    ev1, cpt1 = _pad_edges(adj_index, adj_values)
    ev2, cpt2 = _pad_edges(uadj_index, uadj_values)

    e0 = jnp.concatenate([uEmbeds0, iEmbeds0], axis=0)
    e0 = jnp.pad(e0, ((0, n_pad1 - n_total), (0, 0)))
    e1 = _spmm(ev1, cpt1, e0, n_pad1)
    e2 = _spmm(ev1, cpt1, e1, n_pad1)
    pooled = e0 + e1 + e2

    u0 = jnp.pad(uEmbeds0, ((0, n_pad2 - n_user), (0, 0)))
    u1 = _spmm(ev2, cpt2, u0, n_pad2)
    u2 = _spmm(ev2, cpt2, u1, n_pad2)
    uu = u0 + u1 + u2

    ui_uEmbed = pooled[:n_user]
    ui_iEmbed = pooled[n_user:n_total]
    return (ui_uEmbed, ui_iEmbed, uu[:n_user])


# speedup vs baseline: 1.9489x; 1.1752x over previous
import functools

import jax
import jax.numpy as jnp
from jax import lax
from jax.experimental import pallas as pl
from jax.experimental.pallas import tpu as pltpu
from jax.experimental.pallas import tpu_sc as plsc

N_CORES = 2
N_SUB = 16
N_TILES = N_CORES * N_SUB
CHUNK = 112
NVR = CHUNK // 16
LAT = 128
ZR = 16
NBUF = 3


def _spmm_body(n_pad, cpt, ev_hbm, x_hbm, out_hbm, *scratch):
    evs = list(scratch[:NBUF])
    msgs = list(scratch[NBUF:2 * NBUF])
    zbuf = scratch[2 * NBUF]
    acc = scratch[2 * NBUF + 1]
    semG = list(scratch[2 * NBUF + 2:2 * NBUF + 2 + NBUF])
    semS = list(scratch[2 * NBUF + 2 + NBUF:])

    c_ax = lax.axis_index("c")
    s_ax = lax.axis_index("s")
    tile = c_ax * N_SUB + s_ax
    chunk_lo = tile * cpt

    z16 = jnp.zeros((16,), jnp.float32)

    def zb_body(k, _):
        zbuf[k // 8, pl.ds((k % 8) * 16, 16)] = z16
        return 0

    lax.fori_loop(0, ZR * 8, zb_body, 0)

    rpt = n_pad // N_SUB
    row_lo = s_ax * rpt
    off = 0
    while off < rpt:
        sz = min(ZR, rpt - off)
        pltpu.sync_copy(zbuf.at[pl.ds(0, sz)], acc.at[pl.ds(row_lo + off, sz)])
        off += sz
    plsc.subcore_barrier()

    iota16 = lax.iota(jnp.int32, 16)
    rowregs = [iota16 + 16 * g for g in range(NVR)]

    def scale_chunk(buf, ev):
        valregs = [plsc.bitcast(ev[2, pl.ds(16 * g, 16)], jnp.float32)
                   for g in range(NVR)]

        @plsc.parallel_loop(0, LAT, 1, unroll=4)
        def col_body(col):
            colidx = jnp.full((16,), col, jnp.int32)
            ms = [plsc.load_gather(buf, [rowregs[g], colidx])
                  for g in range(NVR)]
            for g in range(NVR):
                plsc.store_scatter(buf, [rowregs[g], colidx],
                                   ms[g] * valregs[g])

    def fire_gather(slot, c):
        pltpu.sync_copy(ev_hbm.at[chunk_lo + c], evs[slot])
        pltpu.async_copy(x_hbm.at[evs[slot].at[0]], msgs[slot], semG[slot])

    def wait_scatter(slot):
        pltpu.make_async_copy(
            msgs[slot], acc.at[evs[slot].at[1]], semS[slot]).wait()

    def do_chunk(b):
        pltpu.make_async_copy(x_hbm.at[evs[b].at[0]], msgs[b], semG[b]).wait()
        scale_chunk(msgs[b], evs[b])
        pltpu.async_copy(msgs[b], acc.at[evs[b].at[1]], semS[b], add=True)

    D = NBUF - 1
    G = cpt // NBUF

    for b in range(D):
        fire_gather(b, b)

    for b in range(NBUF):
        p = (b + D) % NBUF
        if b + D >= NBUF:
            wait_scatter(p)
        fire_gather(p, b + D)
        do_chunk(b)

    def group_body(g, _):
        c0 = g * NBUF
        for b in range(NBUF):
            p = (b + D) % NBUF
            wait_scatter(p)
            fire_gather(p, c0 + b + D)
            do_chunk(b)
        return 0

    lax.fori_loop(1, G - 1, group_body, 0)

    c0 = (G - 1) * NBUF
    for b in range(NBUF):
        if b + D < NBUF:
            p = b + D
            wait_scatter(p)
            fire_gather(p, c0 + b + D)
        do_chunk(b)
    for b in range(NBUF):
        wait_scatter(b)
    plsc.subcore_barrier()

    off = 0
    while off < rpt:
        sz = min(ZR, rpt - off)
        pltpu.sync_copy(acc.at[pl.ds(row_lo + off, sz)], zbuf.at[pl.ds(0, sz)])
        pltpu.sync_copy(zbuf.at[pl.ds(0, sz)],
                        out_hbm.at[c_ax, pl.ds(row_lo + off, sz)])
        off += sz


@functools.partial(jax.jit, static_argnames=("n_pad", "cpt"))
def _spmm_partials(ev, x, *, n_pad, cpt):
    mesh = plsc.VectorSubcoreMesh(core_axis_name="c", subcore_axis_name="s")
    body = functools.partial(_spmm_body, n_pad, cpt)
    kern = pl.kernel(
        body,
        out_type=jax.ShapeDtypeStruct((N_CORES, n_pad, LAT), jnp.float32),
        mesh=mesh,
        compiler_params=pltpu.CompilerParams(needs_layout_passes=False),
        scratch_types=(
            [pltpu.VMEM((3, CHUNK), jnp.int32) for _ in range(NBUF)] +
            [pltpu.VMEM((CHUNK, LAT), jnp.float32) for _ in range(NBUF)] +
            [pltpu.VMEM((ZR, LAT), jnp.float32),
             pltpu.VMEM_SHARED((n_pad, LAT), jnp.float32)] +
            [pltpu.SemaphoreType.DMA for _ in range(2 * NBUF)]
        ),
    )
    return kern(ev, x)


def _pad_edges(index, values):
    e = values.shape[0]
    blk = N_TILES * CHUNK * NBUF
    ep = -(-e // blk) * blk
    pad = ep - e
    cpt = ep // (N_TILES * CHUNK)
    nch = N_TILES * cpt
    cols = jnp.pad(index[1], (0, pad)).reshape(nch, 1, CHUNK)
    rows = jnp.pad(index[0], (0, pad)).reshape(nch, 1, CHUNK)
    vbits = lax.bitcast_convert_type(jnp.pad(values, (0, pad)),
                                     jnp.int32).reshape(nch, 1, CHUNK)
    ev = jnp.concatenate([cols, rows, vbits], axis=1)
    return ev, cpt


def _spmm(ev, cpt, x, n_pad):
    parts = _spmm_partials(ev, x, n_pad=n_pad, cpt=cpt)
    return parts[0] + parts[1]


def kernel(adj_index, adj_values, uadj_index, uadj_values, uEmbeds0, iEmbeds0):
    n_user = uEmbeds0.shape[0]
    n_item = iEmbeds0.shape[0]
    n_total = n_user + n_item
    n_pad1 = -(-n_total // (N_SUB * 8)) * (N_SUB * 8)
    n_pad2 = -(-n_user // (N_SUB * 8)) * (N_SUB * 8)

    ev1, cpt1 = _pad_edges(adj_index, adj_values)
    ev2, cpt2 = _pad_edges(uadj_index, uadj_values)

    e0 = jnp.concatenate([uEmbeds0, iEmbeds0], axis=0)
    e0 = jnp.pad(e0, ((0, n_pad1 - n_total), (0, 0)))
    e1 = _spmm(ev1, cpt1, e0, n_pad1)
    e2 = _spmm(ev1, cpt1, e1, n_pad1)
    pooled = e0 + e1 + e2

    u0 = jnp.pad(uEmbeds0, ((0, n_pad2 - n_user), (0, 0)))
    u1 = _spmm(ev2, cpt2, u0, n_pad2)
    u2 = _spmm(ev2, cpt2, u1, n_pad2)
    uu = u0 + u1 + u2

    ui_uEmbed = pooled[:n_user]
    ui_iEmbed = pooled[n_user:n_total]
    return (ui_uEmbed, ui_iEmbed, uu[:n_user])


# speedup vs baseline: 2.0729x; 1.0636x over previous
import functools

import jax
import jax.numpy as jnp
from jax import lax
from jax.experimental import pallas as pl
from jax.experimental.pallas import tpu as pltpu
from jax.experimental.pallas import tpu_sc as plsc

N_CORES = 2
N_SUB = 16
N_TILES = N_CORES * N_SUB
LAT = 128
ZR = 16


def _spmm_body(n_pad, cpt, nbuf, chunk, ev_hbm, x_hbm, out_hbm, *scratch):
    evs = list(scratch[:nbuf])
    msgs = list(scratch[nbuf:2 * nbuf])
    zbuf = scratch[2 * nbuf]
    acc = scratch[2 * nbuf + 1]
    semG = list(scratch[2 * nbuf + 2:2 * nbuf + 2 + nbuf])
    semS = list(scratch[2 * nbuf + 2 + nbuf:])
    nvr = chunk // 16

    c_ax = lax.axis_index("c")
    s_ax = lax.axis_index("s")
    tile = c_ax * N_SUB + s_ax
    chunk_lo = tile * cpt

    z16 = jnp.zeros((16,), jnp.float32)

    def zb_body(k, _):
        zbuf[k // 8, pl.ds((k % 8) * 16, 16)] = z16
        return 0

    lax.fori_loop(0, ZR * 8, zb_body, 0)

    rpt = n_pad // N_SUB
    row_lo = s_ax * rpt
    off = 0
    while off < rpt:
        sz = min(ZR, rpt - off)
        pltpu.sync_copy(zbuf.at[pl.ds(0, sz)], acc.at[pl.ds(row_lo + off, sz)])
        off += sz
    plsc.subcore_barrier()

    iota16 = lax.iota(jnp.int32, 16)
    rowregs = [iota16 + 16 * g for g in range(nvr)]

    def scale_chunk(buf, ev):
        valregs = [plsc.bitcast(ev[2, pl.ds(16 * g, 16)], jnp.float32)
                   for g in range(nvr)]

        @plsc.parallel_loop(0, LAT, 1, unroll=4)
        def col_body(col):
            colidx = jnp.full((16,), col, jnp.int32)
            ms = [plsc.load_gather(buf, [rowregs[g], colidx])
                  for g in range(nvr)]
            for g in range(nvr):
                plsc.store_scatter(buf, [rowregs[g], colidx],
                                   ms[g] * valregs[g])

    def fire_gather(slot, c):
        pltpu.sync_copy(ev_hbm.at[chunk_lo + c], evs[slot])
        pltpu.async_copy(x_hbm.at[evs[slot].at[0]], msgs[slot], semG[slot])

    def wait_scatter(slot):
        pltpu.make_async_copy(
            msgs[slot], acc.at[evs[slot].at[1]], semS[slot]).wait()

    def do_chunk(b):
        pltpu.make_async_copy(x_hbm.at[evs[b].at[0]], msgs[b], semG[b]).wait()
        scale_chunk(msgs[b], evs[b])
        pltpu.async_copy(msgs[b], acc.at[evs[b].at[1]], semS[b], add=True)

    D = nbuf - 1
    G = cpt // nbuf

    for b in range(D):
        fire_gather(b, b)

    for b in range(nbuf):
        p = (b + D) % nbuf
        if b + D >= nbuf:
            wait_scatter(p)
        fire_gather(p, b + D)
        do_chunk(b)

    def group_body(g, _):
        c0 = g * nbuf
        for b in range(nbuf):
            p = (b + D) % nbuf
            wait_scatter(p)
            fire_gather(p, c0 + b + D)
            do_chunk(b)
        return 0

    lax.fori_loop(1, G - 1, group_body, 0)

    c0 = (G - 1) * nbuf
    for b in range(nbuf):
        if b + D < nbuf:
            p = b + D
            wait_scatter(p)
            fire_gather(p, c0 + b + D)
        do_chunk(b)
    for b in range(nbuf):
        wait_scatter(b)
    plsc.subcore_barrier()

    off = 0
    while off < rpt:
        sz = min(ZR, rpt - off)
        pltpu.sync_copy(acc.at[pl.ds(row_lo + off, sz)], zbuf.at[pl.ds(0, sz)])
        pltpu.sync_copy(zbuf.at[pl.ds(0, sz)],
                        out_hbm.at[c_ax, pl.ds(row_lo + off, sz)])
        off += sz


@functools.partial(jax.jit, static_argnames=("n_pad", "cpt", "nbuf", "chunk"))
def _spmm_partials(ev, x, *, n_pad, cpt, nbuf, chunk):
    mesh = plsc.VectorSubcoreMesh(core_axis_name="c", subcore_axis_name="s")
    body = functools.partial(_spmm_body, n_pad, cpt, nbuf, chunk)
    kern = pl.kernel(
        body,
        out_type=jax.ShapeDtypeStruct((N_CORES, n_pad, LAT), jnp.float32),
        mesh=mesh,
        compiler_params=pltpu.CompilerParams(needs_layout_passes=False),
        scratch_types=(
            [pltpu.VMEM((3, chunk), jnp.int32) for _ in range(nbuf)] +
            [pltpu.VMEM((chunk, LAT), jnp.float32) for _ in range(nbuf)] +
            [pltpu.VMEM((ZR, LAT), jnp.float32),
             pltpu.VMEM_SHARED((n_pad, LAT), jnp.float32)] +
            [pltpu.SemaphoreType.DMA for _ in range(2 * nbuf)]
        ),
    )
    return kern(ev, x)


def _pad_edges(index, values, nbuf, chunk):
    e = values.shape[0]
    blk = N_TILES * chunk * nbuf
    ep = -(-e // blk) * blk
    pad = ep - e
    cpt = ep // (N_TILES * chunk)
    nch = N_TILES * cpt
    cols = jnp.pad(index[1], (0, pad)).reshape(nch, 1, chunk)
    rows = jnp.pad(index[0], (0, pad)).reshape(nch, 1, chunk)
    vbits = lax.bitcast_convert_type(jnp.pad(values, (0, pad)),
                                     jnp.int32).reshape(nch, 1, chunk)
    ev = jnp.concatenate([cols, rows, vbits], axis=1)
    return ev, cpt


def _spmm(ev, cpt, x, n_pad, nbuf, chunk):
    parts = _spmm_partials(ev, x, n_pad=n_pad, cpt=cpt, nbuf=nbuf, chunk=chunk)
    return parts[0] + parts[1]


NBUF1, CHUNK1 = 4, 80
NBUF2, CHUNK2 = 4, 128


def kernel(adj_index, adj_values, uadj_index, uadj_values, uEmbeds0, iEmbeds0):
    n_user = uEmbeds0.shape[0]
    n_item = iEmbeds0.shape[0]
    n_total = n_user + n_item
    n_pad1 = -(-n_total // (N_SUB * 8)) * (N_SUB * 8)
    n_pad2 = -(-n_user // (N_SUB * 8)) * (N_SUB * 8)

    ev1, cpt1 = _pad_edges(adj_index, adj_values, NBUF1, CHUNK1)
    ev2, cpt2 = _pad_edges(uadj_index, uadj_values, NBUF2, CHUNK2)

    e0 = jnp.concatenate([uEmbeds0, iEmbeds0], axis=0)
    e0 = jnp.pad(e0, ((0, n_pad1 - n_total), (0, 0)))
    e1 = _spmm(ev1, cpt1, e0, n_pad1, NBUF1, CHUNK1)
    e2 = _spmm(ev1, cpt1, e1, n_pad1, NBUF1, CHUNK1)
    pooled = e0 + e1 + e2

    u0 = jnp.pad(uEmbeds0, ((0, n_pad2 - n_user), (0, 0)))
    u1 = _spmm(ev2, cpt2, u0, n_pad2, NBUF2, CHUNK2)
    u2 = _spmm(ev2, cpt2, u1, n_pad2, NBUF2, CHUNK2)
    uu = u0 + u1 + u2

    ui_uEmbed = pooled[:n_user]
    ui_iEmbed = pooled[n_user:n_total]
    return (ui_uEmbed, ui_iEmbed, uu[:n_user])


# speedup vs baseline: 2.1796x; 1.0515x over previous
import functools

import jax
import jax.numpy as jnp
from jax import lax
from jax.experimental import pallas as pl
from jax.experimental.pallas import tpu as pltpu
from jax.experimental.pallas import tpu_sc as plsc

N_CORES = 2
N_SUB = 16
N_TILES = N_CORES * N_SUB
LAT = 128
ZR = 16


def _spmm_body(n_pad, cpt, nbuf, chunk, ev_hbm, x_hbm, out_hbm, *scratch):
    evs = list(scratch[:nbuf])
    msgs = list(scratch[nbuf:2 * nbuf])
    zbuf = scratch[2 * nbuf]
    acc = scratch[2 * nbuf + 1]
    semG = list(scratch[2 * nbuf + 2:2 * nbuf + 2 + nbuf])
    semS = list(scratch[2 * nbuf + 2 + nbuf:])
    nvr = chunk // 16

    c_ax = lax.axis_index("c")
    s_ax = lax.axis_index("s")
    tile = c_ax * N_SUB + s_ax
    chunk_lo = tile * cpt

    z16 = jnp.zeros((16,), jnp.float32)

    def zb_body(k, _):
        zbuf[k // 8, pl.ds((k % 8) * 16, 16)] = z16
        return 0

    lax.fori_loop(0, ZR * 8, zb_body, 0)

    rpt = n_pad // N_SUB
    row_lo = s_ax * rpt
    off = 0
    while off < rpt:
        sz = min(ZR, rpt - off)
        pltpu.sync_copy(zbuf.at[pl.ds(0, sz)], acc.at[pl.ds(row_lo + off, sz)])
        off += sz
    plsc.subcore_barrier()

    iota16 = lax.iota(jnp.int32, 16)
    rowregs = [iota16 + 16 * g for g in range(nvr)]

    def scale_chunk(buf, ev):
        valregs = [plsc.bitcast(ev[2, pl.ds(16 * g, 16)], jnp.float32)
                   for g in range(nvr)]

        @plsc.parallel_loop(0, LAT, 1, unroll=4)
        def col_body(col):
            colidx = jnp.full((16,), col, jnp.int32)
            ms = [plsc.load_gather(buf, [rowregs[g], colidx])
                  for g in range(nvr)]
            for g in range(nvr):
                plsc.store_scatter(buf, [rowregs[g], colidx],
                                   ms[g] * valregs[g])

    def fire_gather(slot, c):
        pltpu.sync_copy(ev_hbm.at[chunk_lo + c], evs[slot])
        pltpu.async_copy(x_hbm.at[evs[slot].at[0]], msgs[slot], semG[slot])

    def wait_scatter(slot):
        pltpu.make_async_copy(
            msgs[slot], acc.at[evs[slot].at[1]], semS[slot]).wait()

    def do_chunk(b):
        pltpu.make_async_copy(x_hbm.at[evs[b].at[0]], msgs[b], semG[b]).wait()
        scale_chunk(msgs[b], evs[b])
        pltpu.async_copy(msgs[b], acc.at[evs[b].at[1]], semS[b], add=True)

    D = nbuf - 1
    G = cpt // nbuf

    for b in range(D):
        fire_gather(b, b)

    for b in range(nbuf):
        p = (b + D) % nbuf
        if b + D >= nbuf:
            wait_scatter(p)
        fire_gather(p, b + D)
        do_chunk(b)

    def group_body(g, _):
        c0 = g * nbuf
        for b in range(nbuf):
            p = (b + D) % nbuf
            wait_scatter(p)
            fire_gather(p, c0 + b + D)
            do_chunk(b)
        return 0

    lax.fori_loop(1, G - 1, group_body, 0)

    c0 = (G - 1) * nbuf
    for b in range(nbuf):
        if b + D < nbuf:
            p = b + D
            wait_scatter(p)
            fire_gather(p, c0 + b + D)
        do_chunk(b)
    for b in range(nbuf):
        wait_scatter(b)
    plsc.subcore_barrier()

    off = 0
    while off < rpt:
        sz = min(ZR, rpt - off)
        pltpu.sync_copy(acc.at[pl.ds(row_lo + off, sz)], zbuf.at[pl.ds(0, sz)])
        pltpu.sync_copy(zbuf.at[pl.ds(0, sz)],
                        out_hbm.at[c_ax, pl.ds(row_lo + off, sz)])
        off += sz


@functools.partial(jax.jit, static_argnames=("n_pad", "cpt", "nbuf", "chunk"))
def _spmm_partials(ev, x, *, n_pad, cpt, nbuf, chunk):
    mesh = plsc.VectorSubcoreMesh(core_axis_name="c", subcore_axis_name="s")
    body = functools.partial(_spmm_body, n_pad, cpt, nbuf, chunk)
    kern = pl.kernel(
        body,
        out_type=jax.ShapeDtypeStruct((N_CORES, n_pad, LAT), jnp.float32),
        mesh=mesh,
        compiler_params=pltpu.CompilerParams(needs_layout_passes=False),
        scratch_types=(
            [pltpu.VMEM((3, chunk), jnp.int32) for _ in range(nbuf)] +
            [pltpu.VMEM((chunk, LAT), jnp.float32) for _ in range(nbuf)] +
            [pltpu.VMEM((ZR, LAT), jnp.float32),
             pltpu.VMEM_SHARED((n_pad, LAT), jnp.float32)] +
            [pltpu.SemaphoreType.DMA for _ in range(2 * nbuf)]
        ),
    )
    return kern(ev, x)


def _pad_edges(index, values, nbuf, chunk):
    e = values.shape[0]
    blk = N_TILES * chunk * nbuf
    ep = -(-e // blk) * blk
    pad = ep - e
    cpt = ep // (N_TILES * chunk)
    nch = N_TILES * cpt
    cols = jnp.pad(index[1], (0, pad)).reshape(nch, 1, chunk)
    rows = jnp.pad(index[0], (0, pad)).reshape(nch, 1, chunk)
    vbits = lax.bitcast_convert_type(jnp.pad(values, (0, pad)),
                                     jnp.int32).reshape(nch, 1, chunk)
    ev = jnp.concatenate([cols, rows, vbits], axis=1)
    return ev, cpt


def _spmm(ev, cpt, x, n_pad, nbuf, chunk):
    parts = _spmm_partials(ev, x, n_pad=n_pad, cpt=cpt, nbuf=nbuf, chunk=chunk)
    return parts[0] + parts[1]


NBUF1, CHUNK1 = 5, 64
NBUF2, CHUNK2 = 8, 64


def kernel(adj_index, adj_values, uadj_index, uadj_values, uEmbeds0, iEmbeds0):
    n_user = uEmbeds0.shape[0]
    n_item = iEmbeds0.shape[0]
    n_total = n_user + n_item
    n_pad1 = -(-n_total // (N_SUB * 8)) * (N_SUB * 8)
    n_pad2 = -(-n_user // (N_SUB * 8)) * (N_SUB * 8)

    ev1, cpt1 = _pad_edges(adj_index, adj_values, NBUF1, CHUNK1)
    ev2, cpt2 = _pad_edges(uadj_index, uadj_values, NBUF2, CHUNK2)

    e0 = jnp.concatenate([uEmbeds0, iEmbeds0], axis=0)
    e0 = jnp.pad(e0, ((0, n_pad1 - n_total), (0, 0)))
    e1 = _spmm(ev1, cpt1, e0, n_pad1, NBUF1, CHUNK1)
    e2 = _spmm(ev1, cpt1, e1, n_pad1, NBUF1, CHUNK1)
    pooled = e0 + e1 + e2

    u0 = jnp.pad(uEmbeds0, ((0, n_pad2 - n_user), (0, 0)))
    u1 = _spmm(ev2, cpt2, u0, n_pad2, NBUF2, CHUNK2)
    u2 = _spmm(ev2, cpt2, u1, n_pad2, NBUF2, CHUNK2)
    uu = u0 + u1 + u2

    ui_uEmbed = pooled[:n_user]
    ui_iEmbed = pooled[n_user:n_total]
    return (ui_uEmbed, ui_iEmbed, uu[:n_user])


# speedup vs baseline: 2.3208x; 1.0648x over previous
import functools

import jax
import jax.numpy as jnp
from jax import lax
from jax.experimental import pallas as pl
from jax.experimental.pallas import tpu as pltpu
from jax.experimental.pallas import tpu_sc as plsc

N_CORES = 2
N_SUB = 16
N_TILES = N_CORES * N_SUB
LAT = 128
ZR = 8


def _spmm_body(n_pad, cpt, nbuf, chunk, ev_hbm, x_hbm, out_hbm, *scratch):
    evs = list(scratch[:nbuf])
    msgs = list(scratch[nbuf:2 * nbuf])
    evg = list(scratch[2 * nbuf:2 * nbuf + 2])
    zbuf = scratch[2 * nbuf + 2]
    acc = scratch[2 * nbuf + 3]
    semG = list(scratch[2 * nbuf + 4:3 * nbuf + 4])
    semS = list(scratch[3 * nbuf + 4:4 * nbuf + 4])
    semE = list(scratch[4 * nbuf + 4:])
    nvr = chunk // 16

    c_ax = lax.axis_index("c")
    s_ax = lax.axis_index("s")
    tile = c_ax * N_SUB + s_ax
    chunk_lo = tile * cpt
    grp_lo = tile * (cpt // nbuf)

    z16 = jnp.zeros((16,), jnp.float32)

    def zb_body(k, _):
        zbuf[k // 8, pl.ds((k % 8) * 16, 16)] = z16
        return 0

    lax.fori_loop(0, ZR * 8, zb_body, 0)

    rpt = n_pad // N_SUB
    row_lo = s_ax * rpt
    off = 0
    while off < rpt:
        sz = min(ZR, rpt - off)
        pltpu.sync_copy(zbuf.at[pl.ds(0, sz)], acc.at[pl.ds(row_lo + off, sz)])
        off += sz
    plsc.subcore_barrier()

    iota16 = lax.iota(jnp.int32, 16)
    rowregs = [iota16 + 16 * g for g in range(nvr)]

    def scale_chunk(buf, ev):
        valregs = [plsc.bitcast(ev[2, pl.ds(16 * g, 16)], jnp.float32)
                   for g in range(nvr)]

        @plsc.parallel_loop(0, LAT, 1, unroll=4)
        def col_body(col):
            colidx = jnp.full((16,), col, jnp.int32)
            ms = [plsc.load_gather(buf, [rowregs[g], colidx])
                  for g in range(nvr)]
            for g in range(nvr):
                plsc.store_scatter(buf, [rowregs[g], colidx],
                                   ms[g] * valregs[g])

    def fire_gather(slot, parity, pos):
        for r in range(3):
            for k in range(nvr):
                evs[slot][r, pl.ds(16 * k, 16)] = \
                    evg[parity][3 * pos + r, pl.ds(16 * k, 16)]
        pltpu.async_copy(x_hbm.at[evs[slot].at[0]], msgs[slot], semG[slot])

    def fire_ev_prefetch(parity, grp):
        pltpu.async_copy(ev_hbm.at[grp_lo + grp], evg[parity], semE[parity])

    def wait_ev(parity):
        pltpu.make_async_copy(ev_hbm.at[grp_lo], evg[parity],
                              semE[parity]).wait()

    def wait_scatter(slot):
        pltpu.make_async_copy(
            msgs[slot], acc.at[evs[slot].at[1]], semS[slot]).wait()

    def do_chunk(b):
        pltpu.make_async_copy(x_hbm.at[evs[b].at[0]], msgs[b], semG[b]).wait()
        scale_chunk(msgs[b], evs[b])
        pltpu.async_copy(msgs[b], acc.at[evs[b].at[1]], semS[b], add=True)

    D = nbuf - 1
    G = cpt // nbuf

    def group(g, pg, first=False, prefetch=True):
        for b in range(nbuf):
            p = (b + D) % nbuf
            if b == 1:
                wait_ev(1 - pg)
                if prefetch:
                    fire_ev_prefetch(pg, g + 2)
            if (not first) or b + D >= nbuf:
                wait_scatter(p)
            if b == 0:
                fire_gather(p, pg, nbuf - 1)
            else:
                fire_gather(p, 1 - pg, b - 1)
            do_chunk(b)

    pltpu.sync_copy(ev_hbm.at[grp_lo], evg[0])
    fire_ev_prefetch(1, 1)
    for b in range(D):
        fire_gather(b, 0, b)

    group(0, 0, first=True)
    npre = (G - 3) % 2
    for g in range(1, 1 + npre):
        group(g, g % 2)
    ms = 1 + npre

    def pair_body(s, _):
        g = ms + 2 * s
        for j in range(2):
            gg = g + j
            c0 = gg * nbuf
            pg = (ms + j) % 2
            for b in range(nbuf):
                p = (b + D) % nbuf
                if b == 1:
                    wait_ev(1 - pg)
                    fire_ev_prefetch(pg, gg + 2)
                wait_scatter(p)
                if b == 0:
                    fire_gather(p, pg, nbuf - 1)
                else:
                    fire_gather(p, 1 - pg, b - 1)
                do_chunk(b)
        return 0

    lax.fori_loop(0, (G - 3 - npre) // 2, pair_body, 0)

    if G >= 3:
        group(G - 2, (G - 2) % 2, prefetch=False)

    pgl = (G - 1) % 2
    for b in range(nbuf):
        if b + D < nbuf:
            p = b + D
            wait_scatter(p)
            fire_gather(p, pgl, nbuf - 1)
        do_chunk(b)
    for b in range(nbuf):
        wait_scatter(b)
    plsc.subcore_barrier()

    off = 0
    while off < rpt:
        sz = min(ZR, rpt - off)
        pltpu.sync_copy(acc.at[pl.ds(row_lo + off, sz)], zbuf.at[pl.ds(0, sz)])
        pltpu.sync_copy(zbuf.at[pl.ds(0, sz)],
                        out_hbm.at[c_ax, pl.ds(row_lo + off, sz)])
        off += sz


@functools.partial(jax.jit, static_argnames=("n_pad", "cpt", "nbuf", "chunk"))
def _spmm_partials(ev, x, *, n_pad, cpt, nbuf, chunk):
    mesh = plsc.VectorSubcoreMesh(core_axis_name="c", subcore_axis_name="s")
    body = functools.partial(_spmm_body, n_pad, cpt, nbuf, chunk)
    kern = pl.kernel(
        body,
        out_type=jax.ShapeDtypeStruct((N_CORES, n_pad, LAT), jnp.float32),
        mesh=mesh,
        compiler_params=pltpu.CompilerParams(needs_layout_passes=False),
        scratch_types=(
            [pltpu.VMEM((3, chunk), jnp.int32) for _ in range(nbuf)] +
            [pltpu.VMEM((chunk, LAT), jnp.float32) for _ in range(nbuf)] +
            [pltpu.VMEM((nbuf * 3, chunk), jnp.int32) for _ in range(2)] +
            [pltpu.VMEM((ZR, LAT), jnp.float32),
             pltpu.VMEM_SHARED((n_pad, LAT), jnp.float32)] +
            [pltpu.SemaphoreType.DMA for _ in range(2 * nbuf + 2)]
        ),
    )
    return kern(ev, x)


def _pad_edges(index, values, nbuf, chunk):
    e = values.shape[0]
    blk = N_TILES * chunk * nbuf
    ep = -(-e // blk) * blk
    pad = ep - e
    cpt = ep // (N_TILES * chunk)
    nch = N_TILES * cpt
    cols = jnp.pad(index[1], (0, pad)).reshape(nch, 1, chunk)
    rows = jnp.pad(index[0], (0, pad)).reshape(nch, 1, chunk)
    vbits = lax.bitcast_convert_type(jnp.pad(values, (0, pad)),
                                     jnp.int32).reshape(nch, 1, chunk)
    ev = jnp.concatenate([cols, rows, vbits], axis=1)
    return ev.reshape(nch // nbuf, nbuf * 3, chunk), cpt


def _spmm(ev, cpt, x, n_pad, nbuf, chunk):
    parts = _spmm_partials(ev, x, n_pad=n_pad, cpt=cpt, nbuf=nbuf, chunk=chunk)
    return parts[0] + parts[1]


NBUF1, CHUNK1 = 5, 64
NBUF2, CHUNK2 = 8, 64


def kernel(adj_index, adj_values, uadj_index, uadj_values, uEmbeds0, iEmbeds0):
    n_user = uEmbeds0.shape[0]
    n_item = iEmbeds0.shape[0]
    n_total = n_user + n_item
    n_pad1 = -(-n_total // (N_SUB * 8)) * (N_SUB * 8)
    n_pad2 = -(-n_user // (N_SUB * 8)) * (N_SUB * 8)

    ev1, cpt1 = _pad_edges(adj_index, adj_values, NBUF1, CHUNK1)
    ev2, cpt2 = _pad_edges(uadj_index, uadj_values, NBUF2, CHUNK2)

    e0 = jnp.concatenate([uEmbeds0, iEmbeds0], axis=0)
    e0 = jnp.pad(e0, ((0, n_pad1 - n_total), (0, 0)))
    e1 = _spmm(ev1, cpt1, e0, n_pad1, NBUF1, CHUNK1)
    e2 = _spmm(ev1, cpt1, e1, n_pad1, NBUF1, CHUNK1)
    pooled = e0 + e1 + e2

    u0 = jnp.pad(uEmbeds0, ((0, n_pad2 - n_user), (0, 0)))
    u1 = _spmm(ev2, cpt2, u0, n_pad2, NBUF2, CHUNK2)
    u2 = _spmm(ev2, cpt2, u1, n_pad2, NBUF2, CHUNK2)
    uu = u0 + u1 + u2

    ui_uEmbed = pooled[:n_user]
    ui_iEmbed = pooled[n_user:n_total]
    return (ui_uEmbed, ui_iEmbed, uu[:n_user])
